# Initial kernel scaffold; baseline (speedup 1.0000x reference)
import jax
import jax.numpy as jnp
from jax.experimental import pallas as pl


def kernel(x, table):
    raise NotImplementedError("write your pallas kernel here")


# speedup vs baseline: 1.5976x; 1.5976x over previous
import functools

import jax
import jax.numpy as jnp
from jax import lax
from jax.experimental import pallas as pl
from jax.experimental.pallas import tpu as pltpu
from jax.experimental.pallas import tpu_sc as plsc

B = 4
T = 8192
E = 1024

_info = plsc.get_sparse_core_info()
_NC = _info.num_cores
_NS = _info.num_subcores
_NW = _NC * _NS
_ROWS_PER_W = T // _NW
_CHUNK = 32
_NCHUNK = _ROWS_PER_W // _CHUNK
_NBUF = 2

_mesh = plsc.VectorSubcoreMesh(core_axis_name="c", subcore_axis_name="s")


@functools.partial(
    pl.kernel,
    mesh=_mesh,
    out_type=jax.ShapeDtypeStruct((B, T, E), jnp.float32),
    scratch_types=[
        pltpu.VMEM((_NBUF, _CHUNK, E), jnp.float32),
        pltpu.SemaphoreType.DMA((_NBUF,)),
        pltpu.SemaphoreType.DMA((_NBUF,)),
    ],
)
def _broadcast_rows(table_hbm, out_hbm, buf, rsem, wsem):
    wid = lax.axis_index("s") * _NC + lax.axis_index("c")
    base = wid * _ROWS_PER_W

    def read_copy(c):
        k = c % _NBUF
        return pltpu.make_async_copy(
            table_hbm.at[pl.ds(base + c * _CHUNK, _CHUNK)],
            buf.at[k],
            rsem.at[k],
        )

    def write_copy(c, b):
        k = c % _NBUF
        return pltpu.make_async_copy(
            buf.at[k],
            out_hbm.at[b, pl.ds(base + c * _CHUNK, _CHUNK)],
            wsem.at[k],
        )

    read_copy(0).start()
    for c in range(_NCHUNK):
        read_copy(c).wait()
        if c + 1 < _NCHUNK:
            if c >= 1:
                for b in range(B):
                    write_copy(c - 1, b).wait()
            read_copy(c + 1).start()
        for b in range(B):
            write_copy(c, b).start()
    for c in (_NCHUNK - 2, _NCHUNK - 1):
        if c >= 0:
            for b in range(B):
                write_copy(c, b).wait()


def kernel(x, table):
    del x
    return _broadcast_rows(table)


# speedup vs baseline: 1.6118x; 1.0089x over previous
import functools

import jax
import jax.numpy as jnp
from jax import lax
from jax.experimental import pallas as pl
from jax.experimental.pallas import tpu as pltpu
from jax.experimental.pallas import tpu_sc as plsc

B = 4
T = 8192
E = 1024

_info = plsc.get_sparse_core_info()
_NC = _info.num_cores
_NS = _info.num_subcores
_NW = _NC * _NS
_ROWS_PER_W = T // _NW
_CHUNK = 32
_NCHUNK = _ROWS_PER_W // _CHUNK
_NBUF = 3

_mesh = plsc.VectorSubcoreMesh(core_axis_name="c", subcore_axis_name="s")


@functools.partial(
    pl.kernel,
    mesh=_mesh,
    out_type=jax.ShapeDtypeStruct((B, T, E), jnp.float32),
    scratch_types=[
        pltpu.VMEM((_NBUF, _CHUNK, E), jnp.float32),
        pltpu.SemaphoreType.DMA((_NBUF,)),
        pltpu.SemaphoreType.DMA((_NBUF,)),
    ],
)
def _broadcast_rows(table_hbm, out_hbm, buf, rsem, wsem):
    wid = lax.axis_index("s") * _NC + lax.axis_index("c")
    base = wid * _ROWS_PER_W

    def read_copy(c):
        k = c % _NBUF
        return pltpu.make_async_copy(
            table_hbm.at[pl.ds(base + c * _CHUNK, _CHUNK)],
            buf.at[k],
            rsem.at[k],
        )

    def write_copy(c, b):
        k = c % _NBUF
        return pltpu.make_async_copy(
            buf.at[k],
            out_hbm.at[b, pl.ds(base + c * _CHUNK, _CHUNK)],
            wsem.at[k],
        )

    for c in range(min(_NBUF - 1, _NCHUNK)):
        read_copy(c).start()
    for c in range(_NCHUNK):
        read_copy(c).wait()
        nxt = c + _NBUF - 1
        if nxt < _NCHUNK:
            if c >= 1:
                for b in range(B):
                    write_copy(c - 1, b).wait()
            read_copy(nxt).start()
        for b in range(B):
            write_copy(c, b).start()
    for c in range(max(0, _NCHUNK - _NBUF), _NCHUNK):
        for b in range(B):
            write_copy(c, b).wait()


def kernel(x, table):
    del x
    return _broadcast_rows(table)
